# baseline (speedup 1.0000x reference)
import functools

import jax
import jax.numpy as jnp
from jax.experimental import pallas as pl
from jax.experimental.pallas import tpu as pltpu

B, T, D, H = 32, 64, 512, 512
G = 4 * H


def _lstm_kernel(qT_ref, wih_ref, whh_ref, b_ref, hT_ref,
                 seq_ref, gates_ref, h_ref, c_ref):
    layer = pl.program_id(0)

    @pl.when(layer == 0)
    def _():
        seq_ref[...] = qT_ref[...]

    gates_ref[...] = (
        jnp.dot(seq_ref[...], wih_ref[0], preferred_element_type=jnp.float32)
        + b_ref[0]
    ).astype(gates_ref.dtype)

    h_ref[...] = jnp.zeros_like(h_ref)
    c_ref[...] = jnp.zeros_like(c_ref)

    def step(t, carry):
        h = h_ref[...]
        w = whh_ref[0]
        g_i = gates_ref[pl.ds(t * B, B), 0:H] + jnp.dot(
            h, w[:, 0:H], preferred_element_type=jnp.float32)
        g_f = gates_ref[pl.ds(t * B, B), H:2 * H] + jnp.dot(
            h, w[:, H:2 * H], preferred_element_type=jnp.float32)
        g_g = gates_ref[pl.ds(t * B, B), 2 * H:3 * H] + jnp.dot(
            h, w[:, 2 * H:3 * H], preferred_element_type=jnp.float32)
        i = jax.nn.sigmoid(g_i)
        f = jax.nn.sigmoid(g_f)
        gg = jnp.tanh(g_g)
        g_o = gates_ref[pl.ds(t * B, B), 3 * H:4 * H] + jnp.dot(
            h, w[:, 3 * H:4 * H], preferred_element_type=jnp.float32)
        c = f * c_ref[...] + i * gg
        o = jax.nn.sigmoid(g_o)
        h_new = o * jnp.tanh(c)
        c_ref[...] = c
        h_ref[...] = h_new.astype(seq_ref.dtype)
        seq_ref[pl.ds(t * B, B), :] = h_new.astype(seq_ref.dtype)
        return carry

    jax.lax.fori_loop(0, T, step, 0, unroll=8)
    hT_ref[...] = seq_ref[pl.ds((T - 1) * B, B), :].astype(jnp.float32)


def _tail_kernel(img_ref, wi_ref, bi_ref, hT_ref, wq_ref, bq_ref,
                 wc1_ref, bc1_ref, wc2_ref, bc2_ref, out_ref):
    im = jnp.tanh(
        jnp.dot(img_ref[...], wi_ref[...], preferred_element_type=jnp.float32)
        + bi_ref[...])
    q = jnp.tanh(
        jnp.dot(hT_ref[...], wq_ref[...], preferred_element_type=jnp.float32)
        + bq_ref[...])
    f = im * q
    x = jnp.tanh(
        jnp.dot(f, wc1_ref[...], preferred_element_type=jnp.float32)
        + bc1_ref[...])
    out_ref[...] = jnp.tanh(
        jnp.dot(x, wc2_ref[...], preferred_element_type=jnp.float32)
        + bc2_ref[...])


@jax.jit
def kernel(image, question, Wih, Whh, bih, bhh, Wi, bi, Wq, bq, Wc1, bc1,
           Wc2, bc2):
    n_layers = Wih.shape[0]

    qT = jnp.transpose(question, (1, 0, 2)).reshape(T * B, D)
    qT = qT.astype(jnp.bfloat16)
    WihT = jnp.transpose(Wih, (0, 2, 1)).astype(jnp.bfloat16)
    ball = (bih + bhh)[:, None, :]
    WhhT = jnp.transpose(Whh, (0, 2, 1)).astype(jnp.bfloat16)

    hT = pl.pallas_call(
        _lstm_kernel,
        grid=(n_layers,),
        in_specs=[
            pl.BlockSpec((T * B, D), lambda l: (0, 0)),
            pl.BlockSpec((1, D, G), lambda l: (l, 0, 0)),
            pl.BlockSpec((1, D, G), lambda l: (l, 0, 0)),
            pl.BlockSpec((1, 1, G), lambda l: (l, 0, 0)),
        ],
        out_specs=pl.BlockSpec((B, H), lambda l: (0, 0)),
        out_shape=jax.ShapeDtypeStruct((B, H), jnp.float32),
        scratch_shapes=[
            pltpu.VMEM((T * B, D), jnp.bfloat16),
            pltpu.VMEM((T * B, G), jnp.bfloat16),
            pltpu.VMEM((B, H), jnp.bfloat16),
            pltpu.VMEM((B, H), jnp.float32),
        ],
    )(qT, WihT, WhhT, ball)

    img_p = jnp.pad(image, ((0, 0), (0, 24)))
    Wi_p = jnp.pad(Wi, ((0, 24), (0, 0)))
    Wc1_p = jnp.pad(Wc1, ((0, 0), (0, 24)))
    bc1_p = jnp.pad(bc1, (0, 24))
    Wc2_p = jnp.pad(Wc2, ((0, 24), (0, 58)))
    bc2_p = jnp.pad(bc2, (0, 58))

    out_p = pl.pallas_call(
        _tail_kernel,
        out_shape=jax.ShapeDtypeStruct((B, 640), jnp.float32),
    )(img_p, Wi_p, bi[None, :], hT, Wq, bq[None, :],
      Wc1_p, bc1_p[None, :], Wc2_p, bc2_p[None, :])

    return out_p[:, :582]


# speedup vs baseline: 1.0036x; 1.0036x over previous
import functools

import jax
import jax.numpy as jnp
from jax.experimental import pallas as pl
from jax.experimental.pallas import tpu as pltpu

B, T, D, H = 32, 64, 512, 512
G = 4 * H


def _lstm_kernel(qT_ref, wih_ref, whh_ref, b_ref, hT_ref,
                 seq_ref, gates_ref, h_ref, c_ref):
    layer = pl.program_id(0)

    @pl.when(layer == 0)
    def _():
        seq_ref[...] = qT_ref[...]

    gates_ref[...] = (
        jnp.dot(seq_ref[...], wih_ref[0], preferred_element_type=jnp.float32)
        + b_ref[0]
    )

    h_ref[...] = jnp.zeros_like(h_ref)
    c_ref[...] = jnp.zeros_like(c_ref)

    def step(t, carry):
        h = h_ref[...]
        w = whh_ref[0]
        g_i = gates_ref[pl.ds(t * B, B), 0:H] + jnp.dot(
            h, w[:, 0:H], preferred_element_type=jnp.float32)
        g_f = gates_ref[pl.ds(t * B, B), H:2 * H] + jnp.dot(
            h, w[:, H:2 * H], preferred_element_type=jnp.float32)
        g_g = gates_ref[pl.ds(t * B, B), 2 * H:3 * H] + jnp.dot(
            h, w[:, 2 * H:3 * H], preferred_element_type=jnp.float32)
        i = jax.nn.sigmoid(g_i)
        f = jax.nn.sigmoid(g_f)
        gg = jnp.tanh(g_g)
        g_o = gates_ref[pl.ds(t * B, B), 3 * H:4 * H] + jnp.dot(
            h, w[:, 3 * H:4 * H], preferred_element_type=jnp.float32)
        c = f * c_ref[...] + i * gg
        o = jax.nn.sigmoid(g_o)
        h_new = o * jnp.tanh(c)
        c_ref[...] = c
        h_ref[...] = h_new.astype(seq_ref.dtype)
        seq_ref[pl.ds(t * B, B), :] = h_new.astype(seq_ref.dtype)
        return carry

    jax.lax.fori_loop(0, T, step, 0, unroll=16)
    hT_ref[...] = seq_ref[pl.ds((T - 1) * B, B), :].astype(jnp.float32)


def _tail_kernel(img_ref, wi_ref, bi_ref, hT_ref, wq_ref, bq_ref,
                 wc1_ref, bc1_ref, wc2_ref, bc2_ref, out_ref):
    im = jnp.tanh(
        jnp.dot(img_ref[...], wi_ref[...], preferred_element_type=jnp.float32)
        + bi_ref[...])
    q = jnp.tanh(
        jnp.dot(hT_ref[...], wq_ref[...], preferred_element_type=jnp.float32)
        + bq_ref[...])
    f = im * q
    x = jnp.tanh(
        jnp.dot(f, wc1_ref[...], preferred_element_type=jnp.float32)
        + bc1_ref[...])
    out_ref[...] = jnp.tanh(
        jnp.dot(x, wc2_ref[...], preferred_element_type=jnp.float32)
        + bc2_ref[...])


@jax.jit
def kernel(image, question, Wih, Whh, bih, bhh, Wi, bi, Wq, bq, Wc1, bc1,
           Wc2, bc2):
    n_layers = Wih.shape[0]

    qT = jnp.transpose(question, (1, 0, 2)).reshape(T * B, D)
    qT = qT.astype(jnp.bfloat16)
    WihT = jnp.transpose(Wih, (0, 2, 1)).astype(jnp.bfloat16)
    ball = (bih + bhh)[:, None, :]
    WhhT = jnp.transpose(Whh, (0, 2, 1)).astype(jnp.bfloat16)

    hT = pl.pallas_call(
        _lstm_kernel,
        grid=(n_layers,),
        in_specs=[
            pl.BlockSpec((T * B, D), lambda l: (0, 0)),
            pl.BlockSpec((1, D, G), lambda l: (l, 0, 0)),
            pl.BlockSpec((1, D, G), lambda l: (l, 0, 0)),
            pl.BlockSpec((1, 1, G), lambda l: (l, 0, 0)),
        ],
        out_specs=pl.BlockSpec((B, H), lambda l: (0, 0)),
        out_shape=jax.ShapeDtypeStruct((B, H), jnp.float32),
        scratch_shapes=[
            pltpu.VMEM((T * B, D), jnp.bfloat16),
            pltpu.VMEM((T * B, G), jnp.float32),
            pltpu.VMEM((B, H), jnp.bfloat16),
            pltpu.VMEM((B, H), jnp.float32),
        ],
    )(qT, WihT, WhhT, ball)

    img_p = jnp.pad(image, ((0, 0), (0, 24)))
    Wi_p = jnp.pad(Wi, ((0, 24), (0, 0)))
    Wc1_p = jnp.pad(Wc1, ((0, 0), (0, 24)))
    bc1_p = jnp.pad(bc1, (0, 24))
    Wc2_p = jnp.pad(Wc2, ((0, 24), (0, 58)))
    bc2_p = jnp.pad(bc2, (0, 58))

    out_p = pl.pallas_call(
        _tail_kernel,
        out_shape=jax.ShapeDtypeStruct((B, 640), jnp.float32),
    )(img_p, Wi_p, bi[None, :], hT, Wq, bq[None, :],
      Wc1_p, bc1_p[None, :], Wc2_p, bc2_p[None, :])

    return out_p[:, :582]


# speedup vs baseline: 1.0039x; 1.0003x over previous
import jax
import jax.numpy as jnp
from jax.experimental import pallas as pl
from jax.experimental.pallas import tpu as pltpu

B, T, D, H = 32, 64, 512, 512
G = 4 * H


def _lstm_kernel(qT_ref, wih_ref, whh_ref, b_ref, hT_ref,
                 seq_ref, gates_ref, h_ref, c_ref):
    layer = pl.program_id(0)

    @pl.when(layer == 0)
    def _():
        seq_ref[...] = qT_ref[...]

    gates_ref[...] = (
        jnp.dot(seq_ref[...], wih_ref[0], preferred_element_type=jnp.float32)
        + b_ref[0]
    )

    h_ref[...] = jnp.zeros_like(h_ref)
    c_ref[...] = jnp.zeros_like(c_ref)

    def step(t, carry):
        h = h_ref[...]
        w = whh_ref[0]
        g_i = gates_ref[pl.ds(t * B, B), 0:H] + jnp.dot(
            h, w[:, 0:H], preferred_element_type=jnp.float32)
        g_f = gates_ref[pl.ds(t * B, B), H:2 * H] + jnp.dot(
            h, w[:, H:2 * H], preferred_element_type=jnp.float32)
        g_g = gates_ref[pl.ds(t * B, B), 2 * H:3 * H] + jnp.dot(
            h, w[:, 2 * H:3 * H], preferred_element_type=jnp.float32)
        i = jax.nn.sigmoid(g_i)
        f = jax.nn.sigmoid(g_f)
        gg = jnp.tanh(g_g)
        g_o = gates_ref[pl.ds(t * B, B), 3 * H:4 * H] + jnp.dot(
            h, w[:, 3 * H:4 * H], preferred_element_type=jnp.float32)
        c = f * c_ref[...] + i * gg
        o = jax.nn.sigmoid(g_o)
        h_new = o * jnp.tanh(c)
        c_ref[...] = c
        h_ref[...] = h_new.astype(seq_ref.dtype)
        seq_ref[pl.ds(t * B, B), :] = h_new.astype(seq_ref.dtype)
        return carry

    jax.lax.fori_loop(0, T, step, 0, unroll=16)
    hT_ref[...] = seq_ref[pl.ds((T - 1) * B, B), :].astype(jnp.float32)


def _tail_kernel(img_ref, wi_ref, bi_ref, hT_ref, wq_ref, bq_ref,
                 wc1_ref, bc1_ref, wc2_ref, bc2_ref, out_ref):
    im = jnp.tanh(
        jnp.dot(img_ref[...], wi_ref[...], preferred_element_type=jnp.float32)
        + bi_ref[...])
    q = jnp.tanh(
        jnp.dot(hT_ref[...], wq_ref[...], preferred_element_type=jnp.float32)
        + bq_ref[...])
    f = im * q
    x = jnp.tanh(
        jnp.dot(f, wc1_ref[...], preferred_element_type=jnp.float32)
        + bc1_ref[...])
    out_ref[...] = jnp.tanh(
        jnp.dot(x, wc2_ref[...], preferred_element_type=jnp.float32)
        + bc2_ref[...])


@jax.jit
def kernel(image, question, Wih, Whh, bih, bhh, Wi, bi, Wq, bq, Wc1, bc1,
           Wc2, bc2):
    n_layers = Wih.shape[0]

    qT = jnp.transpose(question, (1, 0, 2)).reshape(T * B, D)
    qT = qT.astype(jnp.bfloat16)
    WihT = jnp.transpose(Wih, (0, 2, 1)).astype(jnp.bfloat16)
    ball = (bih + bhh)[:, None, :]
    WhhT = jnp.transpose(Whh, (0, 2, 1)).astype(jnp.bfloat16)

    hT = pl.pallas_call(
        _lstm_kernel,
        grid=(n_layers,),
        in_specs=[
            pl.BlockSpec((T * B, D), lambda l: (0, 0)),
            pl.BlockSpec((1, D, G), lambda l: (l, 0, 0)),
            pl.BlockSpec((1, D, G), lambda l: (l, 0, 0)),
            pl.BlockSpec((1, 1, G), lambda l: (l, 0, 0)),
        ],
        out_specs=pl.BlockSpec((B, H), lambda l: (0, 0)),
        out_shape=jax.ShapeDtypeStruct((B, H), jnp.float32),
        scratch_shapes=[
            pltpu.VMEM((T * B, D), jnp.bfloat16),
            pltpu.VMEM((T * B, G), jnp.float32),
            pltpu.VMEM((B, H), jnp.bfloat16),
            pltpu.VMEM((B, H), jnp.float32),
        ],
    )(qT, WihT, WhhT, ball)

    img_p = jnp.pad(image, ((0, 0), (0, 24)))
    Wi_p = jnp.pad(Wi, ((0, 24), (0, 0)))
    Wc1_p = jnp.pad(Wc1, ((0, 0), (0, 24)))
    bc1_p = jnp.pad(bc1, (0, 24))
    Wc2_p = jnp.pad(Wc2, ((0, 24), (0, 58)))
    bc2_p = jnp.pad(bc2, (0, 58))

    out_p = pl.pallas_call(
        _tail_kernel,
        out_shape=jax.ShapeDtypeStruct((B, 640), jnp.float32),
    )(img_p, Wi_p, bi[None, :], hT, Wq, bq[None, :],
      Wc1_p, bc1_p[None, :], Wc2_p, bc2_p[None, :])

    return out_p[:, :582]
